# baseline (speedup 1.0000x reference)
import functools

import jax
import jax.numpy as jnp
from jax import lax
from jax.experimental import pallas as pl
from jax.experimental.pallas import tpu as pltpu
from jax.experimental.pallas import tpu_sc as plsc

N, E, D, ED, H, L = 10000, 320000, 128, 16, 128, 3
LANES = 16
NC, NS = 2, 16
NPAD = 10240
NW = NC * NS
EPT = E // NW
C = 80
ITERS = EPT // C


def _proj_body(x_ref, w_ref, b_ref, o_ref):
    o_ref[...] = (
        jax.lax.dot_general(x_ref[...], w_ref[...], (((1,), (0,)), ((), ())),
                            preferred_element_type=jnp.float32,
                            precision=jax.lax.Precision.HIGHEST)
        + b_ref[...]
    )


def _proj(x, w, b, blk):
    n = x.shape[0]
    return pl.pallas_call(
        _proj_body,
        grid=(n // blk,),
        in_specs=[
            pl.BlockSpec((blk, x.shape[1]), lambda i: (i, jnp.int32(0))),
            pl.BlockSpec(w.shape, lambda i: (jnp.int32(0), jnp.int32(0))),
            pl.BlockSpec((1, b.shape[1]), lambda i: (jnp.int32(0), jnp.int32(0))),
        ],
        out_specs=pl.BlockSpec((blk, w.shape[1]), lambda i: (i, jnp.int32(0))),
        out_shape=jax.ShapeDtypeStruct((n, w.shape[1]), jnp.float32),
    )(x, w, b)


def _eproj_body(x_ref, w_ref, b_ref, o_ref):
    y = (jax.lax.dot_general(x_ref[...], w_ref[...], (((1,), (0,)), ((), ())),
                             preferred_element_type=jnp.float32,
                             precision=jax.lax.Precision.HIGHEST)
         + b_ref[...])
    blk = y.shape[0]
    bits = jax.lax.bitcast_convert_type(y, jnp.uint32)
    rnd = bits + jnp.uint32(0x7FFF) + ((bits >> 16) & jnp.uint32(1))
    u = rnd >> 16
    u4 = u.reshape(blk, H // 32, 2, 16)
    w = u4[:, :, 0, :] | (u4[:, :, 1, :] << 16)
    o_ref[...] = jax.lax.bitcast_convert_type(w.reshape(blk, H // 2),
                                              jnp.int32)


def _eproj(x, w, b, blk):
    n = x.shape[0]
    return pl.pallas_call(
        _eproj_body,
        grid=(n // blk,),
        in_specs=[
            pl.BlockSpec((blk, x.shape[1]), lambda i: (i, jnp.int32(0))),
            pl.BlockSpec(w.shape, lambda i: (jnp.int32(0), jnp.int32(0))),
            pl.BlockSpec((1, b.shape[1]), lambda i: (jnp.int32(0), jnp.int32(0))),
        ],
        out_specs=pl.BlockSpec((blk, w.shape[1] // 2),
                               lambda i: (i, jnp.int32(0))),
        out_shape=jax.ShapeDtypeStruct((n, w.shape[1] // 2), jnp.int32),
    )(x, w, b)


def _node_body(h_ref, a0_ref, a1_ref, scale_ref, w1_ref, b1_ref, w2_ref,
               b2_ref, g_ref, be_ref, o_ref):
    z = scale_ref[0, 0] * h_ref[...] + a0_ref[...] + a1_ref[...]
    t = jax.lax.dot_general(z, w1_ref[...], (((1,), (0,)), ((), ())),
                            preferred_element_type=jnp.float32,
                            precision=jax.lax.Precision.HIGHEST)
    t = jnp.maximum(t + b1_ref[...], 0.0)
    y = jax.lax.dot_general(t, w2_ref[...], (((1,), (0,)), ((), ())),
                            preferred_element_type=jnp.float32,
                            precision=jax.lax.Precision.HIGHEST)
    y = y + b2_ref[...]
    mu = jnp.mean(y, axis=-1, keepdims=True)
    var = jnp.mean((y - mu) ** 2, axis=-1, keepdims=True)
    y = (y - mu) * jax.lax.rsqrt(var + 1e-5) * g_ref[...] + be_ref[...]
    o_ref[...] = jnp.maximum(y, 0.0)


def _node_update(h, a0, a1, scale, w1, b1, w2, b2, gamma, beta, blk):
    n = h.shape[0]
    return pl.pallas_call(
        _node_body,
        grid=(n // blk,),
        in_specs=[
            pl.BlockSpec((blk, H), lambda i: (i, jnp.int32(0))),
            pl.BlockSpec((blk, H), lambda i: (i, jnp.int32(0))),
            pl.BlockSpec((blk, H), lambda i: (i, jnp.int32(0))),
            pl.BlockSpec((1, 1), lambda i: (jnp.int32(0), jnp.int32(0)),
                         memory_space=pltpu.SMEM),
            pl.BlockSpec((H, 2 * H), lambda i: (jnp.int32(0), jnp.int32(0))),
            pl.BlockSpec((1, 2 * H), lambda i: (jnp.int32(0), jnp.int32(0))),
            pl.BlockSpec((2 * H, H), lambda i: (jnp.int32(0), jnp.int32(0))),
            pl.BlockSpec((1, H), lambda i: (jnp.int32(0), jnp.int32(0))),
            pl.BlockSpec((1, H), lambda i: (jnp.int32(0), jnp.int32(0))),
            pl.BlockSpec((1, H), lambda i: (jnp.int32(0), jnp.int32(0))),
        ],
        out_specs=pl.BlockSpec((blk, H), lambda i: (i, jnp.int32(0))),
        out_shape=jax.ShapeDtypeStruct((n, H), jnp.float32),
    )(h, a0, a1, scale, w1, b1, w2, b2, gamma, beta)


C = 80
ITERS = EPT // C


@functools.cache
def _build_agg():
    return functools.partial(
        pl.kernel,
        out_type=jax.ShapeDtypeStruct((NC, NPAD, H), jnp.float32),
        mesh=plsc.VectorSubcoreMesh(core_axis_name="c", subcore_axis_name="s",
                                    num_cores=NC, num_subcores=NS),
        scratch_types=[
            pltpu.VMEM((C,), jnp.int32),
            pltpu.VMEM((C,), jnp.int32),
            pltpu.VMEM((C,), jnp.int32),
            pltpu.VMEM((C,), jnp.int32),
            pltpu.VMEM((C, H // 2), jnp.int32),
            pltpu.VMEM((C, H // 2), jnp.int32),
            pltpu.VMEM((C, H), jnp.float32),
            pltpu.VMEM((C, H), jnp.float32),
            pltpu.VMEM_SHARED((NPAD, H), jnp.float32),
            pltpu.SemaphoreType.DMA,
            pltpu.SemaphoreType.DMA,
            pltpu.SemaphoreType.DMA,
            pltpu.SemaphoreType.DMA,
            pltpu.SemaphoreType.DMA,
            pltpu.SemaphoreType.DMA,
            pltpu.SemaphoreType.DMA,
            pltpu.SemaphoreType.DMA,
            pltpu.SemaphoreType.DMA,
            pltpu.SemaphoreType.DMA,
        ],
    )(_agg_body)


def _agg_body(e_hbm, h_hbm, src_hbm, dst_hbm, zeros_hbm, out_hbm,
              src0, src1, dst0, dst1, e0, e1, g0, g1, agg_sh,
              ss0, ss1, sd0, sd1, se0, se1, sg0, sg1, sc0, sc1):
    c = lax.axis_index("c")
    s = lax.axis_index("s")
    wid = s * NC + c
    srcs, dsts, ebufs, gbufs = (src0, src1), (dst0, dst1), (e0, e1), (g0, g1)
    sss, sds, ses, sgs, scs = ((ss0, ss1), (sd0, sd1), (se0, se1),
                               (sg0, sg1), (sc0, sc1))
    base0 = wid * jnp.int32(EPT)
    bmax = base0 + jnp.int32(EPT - C)

    pltpu.sync_copy(zeros_hbm, agg_sh.at[pl.ds(s * (NPAD // NS), NPAD // NS)])
    plsc.subcore_barrier()

    def bs(x):
        return jnp.minimum(base0 + x * jnp.int32(C), bmax)

    def i_src(x, p):
        pltpu.async_copy(src_hbm.at[pl.ds(bs(x), C)], srcs[p], sss[p])

    def i_dst(x, p):
        pltpu.async_copy(dst_hbm.at[pl.ds(bs(x), C)], dsts[p], sds[p])

    def w_src(p):
        pltpu.make_async_copy(src_hbm.at[pl.ds(0, C)], srcs[p], sss[p]).wait()

    def w_dst(p):
        pltpu.make_async_copy(dst_hbm.at[pl.ds(0, C)], dsts[p], sds[p]).wait()

    def i_e(x, p):
        pltpu.async_copy(e_hbm.at[pl.ds(bs(x), C)], ebufs[p], ses[p])

    def w_e(p):
        pltpu.make_async_copy(e_hbm.at[pl.ds(0, C)], ebufs[p], ses[p]).wait()

    def i_g(p):
        pltpu.async_copy(h_hbm.at[srcs[p]], gbufs[p], sgs[p])

    def w_g(p):
        pltpu.make_async_copy(e_hbm.at[pl.ds(0, C)], gbufs[p], sgs[p]).wait()

    def i_sc(p):
        pltpu.async_copy(gbufs[p], agg_sh.at[dsts[p]], scs[p], add=True)

    def w_sc(p):
        pltpu.make_async_copy(g0, agg_sh.at[pl.ds(0, C)], scs[p]).wait()

    def compute(p):
        gb, eb = gbufs[p], ebufs[p]

        def rowfn(i, cr):
            for jj in range(H // 32):
                wv = eb[i, pl.ds(jj * 16, 16)]
                ea = jax.lax.bitcast_convert_type(wv << 16, jnp.float32)
                eb2 = jax.lax.bitcast_convert_type(
                    wv & jnp.int32(-65536), jnp.float32)
                sla = pl.ds(jj * 32, 16)
                slb = pl.ds(jj * 32 + 16, 16)
                gb[i, sla] = jnp.maximum(gb[i, sla] + ea, 0.0)
                gb[i, slb] = jnp.maximum(gb[i, slb] + eb2, 0.0)
            return cr

        lax.fori_loop(jnp.int32(0), jnp.int32(C), rowfn, jnp.int32(0))

    def phase(x, p, first=False, last=False):
        q = 1 - p
        w_src(q)
        if not first:
            w_sc(q)
        if not last:
            i_g(q)
            i_e(x + 1, q)
        w_e(p)
        w_g(p)
        if not last:
            i_src(x + 2, p)
            i_dst(x + 1, q)
        compute(p)
        w_dst(p)
        i_sc(p)

    i_src(jnp.int32(0), 0)
    i_src(jnp.int32(1), 1)
    i_dst(jnp.int32(0), 0)
    w_src(0)
    i_g(0)
    i_e(jnp.int32(0), 0)

    phase(jnp.int32(0), 0, first=True)
    phase(jnp.int32(1), 1)

    def body(k, cr):
        x = k * jnp.int32(2)
        phase(x, 0)
        phase(x + 1, 1)
        return cr

    lax.fori_loop(jnp.int32(1), jnp.int32(ITERS // 2), body, jnp.int32(0))

    phase(jnp.int32(ITERS - 1), 0, last=True)

    w_sc(0)
    plsc.subcore_barrier()

    row0 = s * (NPAD // NS)
    pltpu.sync_copy(agg_sh.at[pl.ds(row0, NPAD // NS)],
                    out_hbm.at[c, pl.ds(row0, NPAD // NS)])


def kernel(x, ei, ea, W_proj, b_proj, eps, We, be, Wm1, bm1, Wm2, bm2,
           gamma, beta):
    src = ei[0].astype(jnp.int32)
    dst = ei[1].astype(jnp.int32)
    zeros = jnp.zeros((NPAD // NS, H), jnp.float32)

    h = _proj(x, W_proj, b_proj.reshape(1, H), 1000)

    for l in range(L):
        e = _eproj(ea, We[l], be[l].reshape(1, H), 4000)
        aggp = _build_agg()(e, h, src, dst, zeros)
        scale = (1.0 + eps[l]).reshape(1, 1).astype(jnp.float32)
        h = _node_update(h, aggp[0], aggp[1], scale, Wm1[l],
                         bm1[l].reshape(1, 2 * H), Wm2[l],
                         bm2[l].reshape(1, H), gamma[l].reshape(1, H),
                         beta[l].reshape(1, H), 1000)
    return h


# speedup vs baseline: 4.5498x; 4.5498x over previous
import functools

import jax
import jax.numpy as jnp
from jax import lax
from jax.experimental import pallas as pl
from jax.experimental.pallas import tpu as pltpu
from jax.experimental.pallas import tpu_sc as plsc

N, E, D, ED, H, L = 10000, 320000, 128, 16, 128, 3
LANES = 16
NC, NS = 2, 16
NPAD = 10240
NW = NC * NS
EPT = E // NW
C = 80
ITERS = EPT // C


def _proj_body(x_ref, w_ref, b_ref, o_ref):
    o_ref[...] = (
        jax.lax.dot_general(x_ref[...], w_ref[...], (((1,), (0,)), ((), ())),
                            preferred_element_type=jnp.float32,
                            precision=jax.lax.Precision.HIGHEST)
        + b_ref[...]
    )


def _proj(x, w, b, blk):
    n = x.shape[0]
    return pl.pallas_call(
        _proj_body,
        grid=(n // blk,),
        in_specs=[
            pl.BlockSpec((blk, x.shape[1]), lambda i: (i, jnp.int32(0))),
            pl.BlockSpec(w.shape, lambda i: (jnp.int32(0), jnp.int32(0))),
            pl.BlockSpec((1, b.shape[1]), lambda i: (jnp.int32(0), jnp.int32(0))),
        ],
        out_specs=pl.BlockSpec((blk, w.shape[1]), lambda i: (i, jnp.int32(0))),
        out_shape=jax.ShapeDtypeStruct((n, w.shape[1]), jnp.float32),
    )(x, w, b)


def _node_body(h_ref, a0_ref, a1_ref, scale_ref, w1_ref, b1_ref, w2_ref,
               b2_ref, g_ref, be_ref, o_ref):
    z = scale_ref[0, 0] * h_ref[...] + a0_ref[...] + a1_ref[...]
    t = jax.lax.dot_general(z, w1_ref[...], (((1,), (0,)), ((), ())),
                            preferred_element_type=jnp.float32,
                            precision=jax.lax.Precision.HIGHEST)
    t = jnp.maximum(t + b1_ref[...], 0.0)
    y = jax.lax.dot_general(t, w2_ref[...], (((1,), (0,)), ((), ())),
                            preferred_element_type=jnp.float32,
                            precision=jax.lax.Precision.HIGHEST)
    y = y + b2_ref[...]
    mu = jnp.mean(y, axis=-1, keepdims=True)
    var = jnp.mean((y - mu) ** 2, axis=-1, keepdims=True)
    y = (y - mu) * jax.lax.rsqrt(var + 1e-5) * g_ref[...] + be_ref[...]
    o_ref[...] = jnp.maximum(y, 0.0)


def _node_update(h, a0, a1, scale, w1, b1, w2, b2, gamma, beta, blk):
    n = h.shape[0]
    return pl.pallas_call(
        _node_body,
        grid=(n // blk,),
        in_specs=[
            pl.BlockSpec((blk, H), lambda i: (i, jnp.int32(0))),
            pl.BlockSpec((blk, H), lambda i: (i, jnp.int32(0))),
            pl.BlockSpec((blk, H), lambda i: (i, jnp.int32(0))),
            pl.BlockSpec((1, 1), lambda i: (jnp.int32(0), jnp.int32(0)),
                         memory_space=pltpu.SMEM),
            pl.BlockSpec((H, 2 * H), lambda i: (jnp.int32(0), jnp.int32(0))),
            pl.BlockSpec((1, 2 * H), lambda i: (jnp.int32(0), jnp.int32(0))),
            pl.BlockSpec((2 * H, H), lambda i: (jnp.int32(0), jnp.int32(0))),
            pl.BlockSpec((1, H), lambda i: (jnp.int32(0), jnp.int32(0))),
            pl.BlockSpec((1, H), lambda i: (jnp.int32(0), jnp.int32(0))),
            pl.BlockSpec((1, H), lambda i: (jnp.int32(0), jnp.int32(0))),
        ],
        out_specs=pl.BlockSpec((blk, H), lambda i: (i, jnp.int32(0))),
        out_shape=jax.ShapeDtypeStruct((n, H), jnp.float32),
    )(h, a0, a1, scale, w1, b1, w2, b2, gamma, beta)


C = 80
ITERS = EPT // C


@functools.cache
def _build_agg():
    return functools.partial(
        pl.kernel,
        out_type=jax.ShapeDtypeStruct((NC, NPAD, H), jnp.float32),
        mesh=plsc.VectorSubcoreMesh(core_axis_name="c", subcore_axis_name="s",
                                    num_cores=NC, num_subcores=NS),
        scratch_types=[
            pltpu.VMEM((C,), jnp.int32),
            pltpu.VMEM((C,), jnp.int32),
            pltpu.VMEM((C,), jnp.int32),
            pltpu.VMEM((C,), jnp.int32),
            pltpu.VMEM((C, H), jnp.float32),
            pltpu.VMEM((C, H), jnp.float32),
            pltpu.VMEM((C, H), jnp.float32),
            pltpu.VMEM((C, H), jnp.float32),
            pltpu.VMEM_SHARED((NPAD, H), jnp.float32),
            pltpu.SemaphoreType.DMA,
            pltpu.SemaphoreType.DMA,
            pltpu.SemaphoreType.DMA,
            pltpu.SemaphoreType.DMA,
            pltpu.SemaphoreType.DMA,
            pltpu.SemaphoreType.DMA,
            pltpu.SemaphoreType.DMA,
            pltpu.SemaphoreType.DMA,
            pltpu.SemaphoreType.DMA,
            pltpu.SemaphoreType.DMA,
        ],
    )(_agg_body)


def _agg_body(e_hbm, h_hbm, src_hbm, dst_hbm, zeros_hbm, out_hbm,
              src0, src1, dst0, dst1, e0, e1, g0, g1, agg_sh,
              ss0, ss1, sd0, sd1, se0, se1, sg0, sg1, sc0, sc1):
    c = lax.axis_index("c")
    s = lax.axis_index("s")
    wid = s * NC + c
    srcs, dsts, ebufs, gbufs = (src0, src1), (dst0, dst1), (e0, e1), (g0, g1)
    sss, sds, ses, sgs, scs = ((ss0, ss1), (sd0, sd1), (se0, se1),
                               (sg0, sg1), (sc0, sc1))
    base0 = wid * jnp.int32(EPT)
    bmax = base0 + jnp.int32(EPT - C)

    pltpu.sync_copy(zeros_hbm, agg_sh.at[pl.ds(s * (NPAD // NS), NPAD // NS)])
    plsc.subcore_barrier()

    def bs(x):
        return jnp.minimum(base0 + x * jnp.int32(C), bmax)

    def i_src(x, p):
        pltpu.async_copy(src_hbm.at[pl.ds(bs(x), C)], srcs[p], sss[p])

    def i_dst(x, p):
        pltpu.async_copy(dst_hbm.at[pl.ds(bs(x), C)], dsts[p], sds[p])

    def w_src(p):
        pltpu.make_async_copy(src_hbm.at[pl.ds(0, C)], srcs[p], sss[p]).wait()

    def w_dst(p):
        pltpu.make_async_copy(dst_hbm.at[pl.ds(0, C)], dsts[p], sds[p]).wait()

    def i_e(x, p):
        pltpu.async_copy(e_hbm.at[pl.ds(bs(x), C)], ebufs[p], ses[p])

    def w_e(p):
        pltpu.make_async_copy(e_hbm.at[pl.ds(0, C)], ebufs[p], ses[p]).wait()

    def i_g(p):
        pltpu.async_copy(h_hbm.at[srcs[p]], gbufs[p], sgs[p])

    def w_g(p):
        pltpu.make_async_copy(e_hbm.at[pl.ds(0, C)], gbufs[p], sgs[p]).wait()

    def i_sc(p):
        pltpu.async_copy(gbufs[p], agg_sh.at[dsts[p]], scs[p], add=True)

    def w_sc(p):
        pltpu.make_async_copy(g0, agg_sh.at[pl.ds(0, C)], scs[p]).wait()

    def compute(p):
        gb, eb = gbufs[p], ebufs[p]

        def rowfn(i, cr):
            for jj in range(H // LANES):
                sl = pl.ds(jj * LANES, LANES)
                gb[i, sl] = jnp.maximum(gb[i, sl] + eb[i, sl], 0.0)
            return cr

        lax.fori_loop(jnp.int32(0), jnp.int32(C), rowfn, jnp.int32(0))

    def phase(x, p, first=False, last=False):
        q = 1 - p
        w_src(q)
        if not first:
            w_sc(q)
        if not last:
            i_g(q)
            i_e(x + 1, q)
        w_e(p)
        w_g(p)
        if not last:
            i_src(x + 2, p)
            i_dst(x + 1, q)
        compute(p)
        w_dst(p)
        i_sc(p)

    i_src(jnp.int32(0), 0)
    i_src(jnp.int32(1), 1)
    i_dst(jnp.int32(0), 0)
    w_src(0)
    i_g(0)
    i_e(jnp.int32(0), 0)

    phase(jnp.int32(0), 0, first=True)
    phase(jnp.int32(1), 1)

    def body(k, cr):
        x = k * jnp.int32(2)
        phase(x, 0)
        phase(x + 1, 1)
        return cr

    lax.fori_loop(jnp.int32(1), jnp.int32(ITERS // 2), body, jnp.int32(0))

    phase(jnp.int32(ITERS - 1), 0, last=True)

    w_sc(0)
    plsc.subcore_barrier()

    row0 = s * (NPAD // NS)
    pltpu.sync_copy(agg_sh.at[pl.ds(row0, NPAD // NS)],
                    out_hbm.at[c, pl.ds(row0, NPAD // NS)])


def kernel(x, ei, ea, W_proj, b_proj, eps, We, be, Wm1, bm1, Wm2, bm2,
           gamma, beta):
    src = ei[0].astype(jnp.int32)
    dst = ei[1].astype(jnp.int32)
    zeros = jnp.zeros((NPAD // NS, H), jnp.float32)

    h = _proj(x, W_proj, b_proj.reshape(1, H), 1000)

    for l in range(L):
        e = _proj(ea, We[l], be[l].reshape(1, H), 4000)
        aggp = _build_agg()(e, h, src, dst, zeros)
        scale = (1.0 + eps[l]).reshape(1, 1).astype(jnp.float32)
        h = _node_update(h, aggp[0], aggp[1], scale, Wm1[l],
                         bm1[l].reshape(1, 2 * H), Wm2[l],
                         bm2[l].reshape(1, H), gamma[l].reshape(1, H),
                         beta[l].reshape(1, H), 1000)
    return h
